# baseline (speedup 1.0000x reference)
import functools

import jax
import jax.numpy as jnp
from jax import lax
from jax.experimental import pallas as pl
from jax.experimental.pallas import tpu as pltpu
from jax.experimental.pallas import tpu_sc as plsc

B, L, D = 4096, 200, 64
DP = 128
N = B * L
C = 128
NBUF = 2
LANES = 16
NH = N // 2
BH = B // 2
BBLK = 128
LBLK = 8


def _sc_body(hbase, g_per_w, item_idx, test_idx, tag_idx,
             item_tab, test_tab, tag_tab, pos_tab, out,
             idx_v, rows_v, stage_v, pos_v,
             isem0, isem1, gsem0, gsem1, osem):
    nc = plsc.get_sparse_core_info().num_cores
    wid = lax.axis_index("s") * nc + lax.axis_index("c")
    row0 = wid * g_per_w
    isems = (isem0, isem1)
    gsems = (gsem0, gsem1)
    idx_hbms = (item_idx, test_idx, tag_idx)
    tabs = (item_tab, test_tab, tag_tab)

    pltpu.sync_copy(pos_tab, pos_v)

    def islot(b, t):
        return pl.ds((b * 3 + t) * C, C)

    def issue_idx(g, b):
        base = hbase + (row0 + g) * C
        for t in range(3):
            pltpu.async_copy(idx_hbms[t].at[pl.ds(base, C)],
                             idx_v.at[islot(b, t)], isems[b])

    def wait_idx(b):
        for t in range(3):
            pltpu.make_async_copy(idx_hbms[t].at[pl.ds(0, C)],
                                  idx_v.at[islot(b, t)], isems[b]).wait()

    def issue_gathers(b):
        for t in range(3):
            pltpu.async_copy(tabs[t].at[idx_v.at[islot(b, t)]],
                             rows_v.at[b, t], gsems[b])

    def wait_gathers(b):
        for t in range(3):
            pltpu.make_async_copy(tabs[t].at[idx_v.at[islot(b, t)]],
                                  rows_v.at[b, t], gsems[b]).wait()

    def issue_out(g):
        base = (row0 + g) * C
        pltpu.async_copy(stage_v, out.at[pl.ds(base, C)], osem)

    def wait_out():
        pltpu.make_async_copy(stage_v, out.at[pl.ds(0, C)], osem).wait()

    def compute(g, b):
        ita = rows_v.at[b, 0]
        tst = rows_v.at[b, 1]
        tag = rows_v.at[b, 2]
        pbase = lax.rem(hbase + (row0 + g) * C, L)

        def row(i, p):
            for q in range(D // LANES):
                sl = pl.ds(q * LANES, LANES)
                pv = pos_v[pl.ds(p * D + q * LANES, LANES)]
                stage_v[i, sl] = (ita[i, sl] + tst[i, sl] + tag[i, sl]
                                  + pv) * 0.25
            p = p + 1
            return lax.select(p == L, 0, p)

        lax.fori_loop(0, C, row, pbase, unroll=2)

    issue_idx(0, 0)
    issue_idx(1, 1)
    wait_idx(0)
    issue_gathers(0)

    def step(m, carry):
        for j in range(NBUF):
            g = m * NBUF + j
            nb = (j + 1) % NBUF
            wait_gathers(j)

            @pl.when(g + 2 < g_per_w)
            def _():
                issue_idx(g + 2, j)

            @pl.when(g + 1 < g_per_w)
            def _():
                wait_idx(nb)
                issue_gathers(nb)

            @pl.when(g > 0)
            def _():
                wait_out()

            compute(g, j)
            issue_out(g)
        return carry

    lax.fori_loop(0, g_per_w // NBUF, step, 0, unroll=False)
    wait_out()


def _tp_body(src, dst):
    dst[...] = jnp.transpose(src[...], (1, 2, 0))


def _tp_body_alias(src, _prev, dst):
    dst[...] = jnp.transpose(src[...], (1, 2, 0))


def _tc_transpose(half, prev, koff):
    o3 = half.reshape(BH, L, D)
    grid = (BH // BBLK, L // LBLK)
    nb = BH // BBLK
    out_spec = pl.BlockSpec(
        (LBLK, D, BBLK), lambda i, j: (j, 0, i + koff * nb))
    out_shape = jax.ShapeDtypeStruct((L, D, B), jnp.float32)
    in_spec = pl.BlockSpec((BBLK, LBLK, D), lambda i, j: (i, j, 0))
    if prev is None:
        return pl.pallas_call(
            _tp_body,
            grid=grid,
            in_specs=[in_spec],
            out_specs=out_spec,
            out_shape=out_shape,
        )(o3)
    return pl.pallas_call(
        _tp_body_alias,
        grid=grid,
        in_specs=[in_spec, pl.BlockSpec(memory_space=pl.ANY)],
        out_specs=out_spec,
        out_shape=out_shape,
        input_output_aliases={1: 0},
    )(o3, prev)


def kernel(item_idx, test_idx, tag_idx, item_table, test_table, tag_table,
           pos_table):
    info = plsc.get_sparse_core_info()
    nw = info.num_cores * info.num_subcores
    g_per_w = NH // (C * nw)

    item2 = item_idx.astype(jnp.int32).reshape(N)
    test2 = test_idx.astype(jnp.int32).reshape(N)
    tag2 = tag_idx.astype(jnp.int32).reshape(N)
    pad = ((0, 0), (0, DP - D))
    itab = jnp.pad(item_table, pad)
    ttab = jnp.pad(test_table, pad)
    gtab = jnp.pad(tag_table, pad)
    pos1 = pos_table.reshape(L * D)

    mesh = plsc.VectorSubcoreMesh(core_axis_name="c", subcore_axis_name="s")

    def run(hbase):
        return pl.kernel(
            out_type=jax.ShapeDtypeStruct((NH, D), jnp.float32),
            mesh=mesh,
            compiler_params=pltpu.CompilerParams(use_tc_tiling_on_sc=True),
            scratch_types=[
                pltpu.VMEM((NBUF * 3 * C,), jnp.int32),
                pltpu.VMEM((NBUF, 3, C, DP), jnp.float32),
                pltpu.VMEM((C, D), jnp.float32),
                pltpu.VMEM((L * D,), jnp.float32),
                pltpu.SemaphoreType.DMA,
                pltpu.SemaphoreType.DMA,
                pltpu.SemaphoreType.DMA,
                pltpu.SemaphoreType.DMA,
                pltpu.SemaphoreType.DMA,
            ],
        )(functools.partial(_sc_body, hbase, g_per_w))

    o0 = run(0)(item2, test2, tag2, itab, ttab, gtab, pos1)
    o1 = run(NH)(item2, test2, tag2, itab, ttab, gtab, pos1)
    t0 = _tc_transpose(o0, None, 0)
    t1 = _tc_transpose(o1, t0, 1)
    return jnp.transpose(t1, (2, 0, 1))


# speedup vs baseline: 2.3293x; 2.3293x over previous
import functools

import jax
import jax.numpy as jnp
from jax import lax
from jax.experimental import pallas as pl
from jax.experimental.pallas import tpu as pltpu
from jax.experimental.pallas import tpu_sc as plsc

B, L, D = 4096, 200, 64
DP = 128
N = B * L
C = 128
NBUF = 2
LANES = 16


def _sc_body(g_per_w, item_idx, test_idx, tag_idx,
             item_tab, test_tab, tag_tab, pos_tab, out,
             idx_v, rows_v, stage_v, pos_v,
             isem0, isem1, gsem0, gsem1, osem):
    nc = plsc.get_sparse_core_info().num_cores
    wid = lax.axis_index("s") * nc + lax.axis_index("c")
    row0 = wid * g_per_w
    isems = (isem0, isem1)
    gsems = (gsem0, gsem1)
    idx_hbms = (item_idx, test_idx, tag_idx)
    tabs = (item_tab, test_tab, tag_tab)

    pltpu.sync_copy(pos_tab, pos_v)

    def islot(b, t):
        return pl.ds((b * 3 + t) * C, C)

    def issue_idx(g, b):
        base = (row0 + g) * C
        for t in range(3):
            pltpu.async_copy(idx_hbms[t].at[pl.ds(base, C)],
                             idx_v.at[islot(b, t)], isems[b])

    def wait_idx(b):
        for t in range(3):
            pltpu.make_async_copy(idx_hbms[t].at[pl.ds(0, C)],
                                  idx_v.at[islot(b, t)], isems[b]).wait()

    def issue_gathers(b):
        for t in range(3):
            pltpu.async_copy(tabs[t].at[idx_v.at[islot(b, t)]],
                             rows_v.at[b, t], gsems[b])

    def wait_gathers(b):
        for t in range(3):
            pltpu.make_async_copy(tabs[t].at[idx_v.at[islot(b, t)]],
                                  rows_v.at[b, t], gsems[b]).wait()

    def issue_out(g):
        base = (row0 + g) * C
        pltpu.async_copy(stage_v, out.at[pl.ds(base, C)], osem)

    def wait_out():
        pltpu.make_async_copy(stage_v, out.at[pl.ds(0, C)], osem).wait()

    def compute(g, b):
        ita = rows_v.at[b, 0]
        tst = rows_v.at[b, 1]
        tag = rows_v.at[b, 2]
        pbase = lax.rem((row0 + g) * C, L)

        def row(i, p):
            for q in range(D // LANES):
                sl = pl.ds(q * LANES, LANES)
                pv = pos_v[pl.ds(p * D + q * LANES, LANES)]
                stage_v[i, sl] = (ita[i, sl] + tst[i, sl] + tag[i, sl]
                                  + pv) * 0.25
            p = p + 1
            return lax.select(p == L, 0, p)

        lax.fori_loop(0, C, row, pbase, unroll=2)

    issue_idx(0, 0)
    issue_idx(1, 1)
    wait_idx(0)
    issue_gathers(0)

    def step(m, carry):
        for j in range(NBUF):
            g = m * NBUF + j
            nb = (j + 1) % NBUF
            wait_gathers(j)

            @pl.when(g + 2 < g_per_w)
            def _():
                issue_idx(g + 2, j)

            @pl.when(g + 1 < g_per_w)
            def _():
                wait_idx(nb)
                issue_gathers(nb)

            @pl.when(g > 0)
            def _():
                wait_out()

            compute(g, j)
            issue_out(g)
        return carry

    lax.fori_loop(0, g_per_w // NBUF, step, 0, unroll=False)
    wait_out()


def kernel(item_idx, test_idx, tag_idx, item_table, test_table, tag_table,
           pos_table):
    info = plsc.get_sparse_core_info()
    nw = info.num_cores * info.num_subcores
    g_per_w = N // (C * nw)

    item2 = item_idx.astype(jnp.int32).reshape(N)
    test2 = test_idx.astype(jnp.int32).reshape(N)
    tag2 = tag_idx.astype(jnp.int32).reshape(N)
    pad = ((0, 0), (0, DP - D))
    itab = jnp.pad(item_table, pad)
    ttab = jnp.pad(test_table, pad)
    gtab = jnp.pad(tag_table, pad)
    pos1 = pos_table.reshape(L * D)

    mesh = plsc.VectorSubcoreMesh(core_axis_name="c", subcore_axis_name="s")
    run = functools.partial(
        pl.kernel,
        out_type=jax.ShapeDtypeStruct((N, D), jnp.float32),
        mesh=mesh,
        compiler_params=pltpu.CompilerParams(use_tc_tiling_on_sc=True),
        scratch_types=[
            pltpu.VMEM((NBUF * 3 * C,), jnp.int32),
            pltpu.VMEM((NBUF, 3, C, DP), jnp.float32),
            pltpu.VMEM((C, D), jnp.float32),
            pltpu.VMEM((L * D,), jnp.float32),
            pltpu.SemaphoreType.DMA,
            pltpu.SemaphoreType.DMA,
            pltpu.SemaphoreType.DMA,
            pltpu.SemaphoreType.DMA,
            pltpu.SemaphoreType.DMA,
        ],
    )(functools.partial(_sc_body, g_per_w))

    out = run(item2, test2, tag2, itab, ttab, gtab, pos1)
    return out.reshape(B, L, D)
